# Initial kernel scaffold; baseline (speedup 1.0000x reference)
import jax
import jax.numpy as jnp
from jax.experimental import pallas as pl


def kernel(feat, edge_index, W, b):
    raise NotImplementedError("write your pallas kernel here")


# speedup vs baseline: 4.2117x; 4.2117x over previous
import functools

import jax
import jax.numpy as jnp
from jax import lax
from jax.experimental import pallas as pl
from jax.experimental.pallas import tpu as pltpu
from jax.experimental.pallas import tpu_sc as plsc

N_NODES = 10000
D = 256
DH = 128
N_SC = 2
N_TILES = 16
CHUNK = 128
ROWS_PER_TILE = 624
TAIL_ROWS = N_NODES - N_TILES * ROWS_PER_TILE
ACC_ROWS = N_NODES + 240


def _sc_aggregate(feat_cat, src_lo, src_hi, dst_idx):
    n_chunks = src_lo.shape[1]
    mesh = plsc.VectorSubcoreMesh(core_axis_name="c", subcore_axis_name="s")

    @functools.partial(
        pl.kernel,
        mesh=mesh,
        out_type=jax.ShapeDtypeStruct((N_SC * N_NODES, DH), jnp.float32),
        scratch_types=[
            pltpu.VMEM_SHARED((ACC_ROWS, DH), jnp.float32),
            pltpu.VMEM((n_chunks, CHUNK), jnp.int32),
            pltpu.VMEM((n_chunks, CHUNK), jnp.int32),
            pltpu.VMEM((CHUNK, DH), jnp.float32),
            pltpu.SemaphoreType.DMA,
        ],
    )
    def agg(feat_hbm, src_lo_hbm, src_hi_hbm, dst_hbm, out_hbm,
            acc, src_v, dst_v, rows_v, sem):
        c = lax.axis_index("c")
        s = lax.axis_index("s")
        node0 = s * ROWS_PER_TILE

        pltpu.sync_copy(
            feat_hbm.at[pl.ds(c * N_NODES + node0, ROWS_PER_TILE)],
            acc.at[pl.ds(node0, ROWS_PER_TILE)])

        @pl.when(s == N_TILES - 1)
        def _():
            tail0 = N_TILES * ROWS_PER_TILE
            pltpu.sync_copy(
                feat_hbm.at[pl.ds(c * N_NODES + tail0, TAIL_ROWS)],
                acc.at[pl.ds(tail0, TAIL_ROWS)])

        @pl.when(c == 0)
        def _():
            pltpu.sync_copy(src_lo_hbm.at[s], src_v)

        @pl.when(c == 1)
        def _():
            pltpu.sync_copy(src_hi_hbm.at[s], src_v)

        pltpu.sync_copy(dst_hbm.at[s], dst_v)
        plsc.subcore_barrier()

        def body(j, carry):
            pltpu.async_copy(feat_hbm.at[src_v.at[j]], rows_v, sem).wait()
            pltpu.sync_copy(rows_v, acc.at[dst_v.at[j]], add=True)
            return carry

        lax.fori_loop(0, n_chunks, body, 0)

        plsc.subcore_barrier()
        pltpu.sync_copy(
            acc.at[pl.ds(node0, ROWS_PER_TILE)],
            out_hbm.at[pl.ds(c * N_NODES + node0, ROWS_PER_TILE)])

        @pl.when(s == N_TILES - 1)
        def _():
            tail0 = N_TILES * ROWS_PER_TILE
            pltpu.sync_copy(
                acc.at[pl.ds(tail0, TAIL_ROWS)],
                out_hbm.at[pl.ds(c * N_NODES + tail0, TAIL_ROWS)])

    return agg(feat_cat, src_lo, src_hi, dst_idx)


def _tc_linear(rst_cat, W, b2):
    MB = 1000
    nblk = N_NODES // MB

    def body(lo_ref, hi_ref, w_ref, b_ref, out_ref):
        w = w_ref[...]
        acc = lax.dot_general(lo_ref[...], w[:, :DH],
                              (((1,), (1,)), ((), ())),
                              preferred_element_type=jnp.float32)
        acc = acc + lax.dot_general(hi_ref[...], w[:, DH:],
                                    (((1,), (1,)), ((), ())),
                                    preferred_element_type=jnp.float32)
        out_ref[...] = acc + b_ref[...]

    return pl.pallas_call(
        body,
        grid=(nblk,),
        in_specs=[
            pl.BlockSpec((MB, DH), lambda i: (i, 0)),
            pl.BlockSpec((MB, DH), lambda i: (i + nblk, 0)),
            pl.BlockSpec((D, D), lambda i: (0, 0)),
            pl.BlockSpec((1, D), lambda i: (0, 0)),
        ],
        out_specs=pl.BlockSpec((MB, D), lambda i: (i, 0)),
        out_shape=jax.ShapeDtypeStruct((N_NODES, D), jnp.float32),
    )(rst_cat, rst_cat, W, b2)


def kernel(feat, edge_index, W, b):
    src = edge_index[0].astype(jnp.int32)
    dst = edge_index[1].astype(jnp.int32)
    e = src.shape[0]
    n_chunks = -(-e // (N_TILES * CHUNK))
    e_pad = N_TILES * n_chunks * CHUNK
    pad = e_pad - e

    feat_cat = jnp.concatenate([feat[:, :DH], feat[:, DH:]], axis=0)

    src_p = jnp.concatenate([src, jnp.zeros((pad,), jnp.int32)])
    dst_p = jnp.concatenate([dst, jnp.full((pad,), N_NODES, jnp.int32)])
    src_lo = src_p.reshape(N_TILES, n_chunks, CHUNK)
    src_hi = src_lo + N_NODES
    dst_r = dst_p.reshape(N_TILES, n_chunks, CHUNK)

    rst_cat = _sc_aggregate(feat_cat, src_lo, src_hi, dst_r)
    return _tc_linear(rst_cat, W, b.reshape(1, D))
